# baseline (speedup 1.0000x reference)
import functools

import jax
import jax.numpy as jnp
from jax import lax
from jax.experimental import pallas as pl
from jax.experimental.pallas import tpu as pltpu
from jax.experimental.pallas import tpu_sc as plsc

E = 1600000
IN_F = 16
OUT_F = 16
NUM_TYPES = 4

BLKE = 16384
NBLK = -(-E // BLKE)


def _tc_body(x_ref, t_ref, w_ref, b_ref, out_ref):
    x = x_ref[...]
    tw = t_ref[...][None, :]
    ys = []
    for tt in range(NUM_TYPES):
        y = lax.dot_general(
            w_ref[tt], x,
            dimension_numbers=(((1,), (0,)), ((), ())),
            preferred_element_type=jnp.float32,
        ) + b_ref[tt][:, None]
        ys.append(y)
    m0 = tw == 0
    m1 = tw == 1
    m2 = tw == 2
    ysel = jnp.where(m0, ys[0], jnp.where(m1, ys[1], jnp.where(m2, ys[2], ys[3])))
    out_ref[...] = jnp.maximum(ysel, 0.0)


SC_ROWS = 40
SC_CHE = SC_ROWS * 128
SC_FULL = E // SC_CHE
SC_TAIL = E - SC_FULL * SC_CHE
SC_NBLK = SC_FULL + 1
NW = 32
SC_NIT = -(-SC_NBLK // NW)


def _sc_impl(xt, types, t_tail, wflat, bflat):
    mesh = plsc.VectorSubcoreMesh(core_axis_name="c", subcore_axis_name="s")

    @functools.partial(
        pl.kernel,
        mesh=mesh,
        out_type=jax.ShapeDtypeStruct((OUT_F, E), jnp.float32),
        compiler_params=pltpu.CompilerParams(needs_layout_passes=False),
        scratch_types=[
            pltpu.VMEM((IN_F, SC_CHE), jnp.float32),
            pltpu.VMEM((SC_CHE,), jnp.int32),
            pltpu.VMEM((NUM_TYPES * OUT_F * IN_F,), jnp.float32),
            pltpu.VMEM((NUM_TYPES * OUT_F,), jnp.float32),
        ],
    )
    def k(xt_hbm, t_hbm, ttail_hbm, w_hbm, b_hbm, out_hbm, x_v, t_v, w_v, b_v):
        wid = lax.axis_index("s") * 2 + lax.axis_index("c")
        pltpu.sync_copy(w_hbm, w_v)
        pltpu.sync_copy(b_hbm, b_v)

        def group_body(g, carry2):
            e0 = g * 16
            tvec = t_v[pl.ds(e0, 16)]
            wb = tvec * (OUT_F * IN_F)
            bb = tvec * OUT_F
            xcols = [x_v[k_, pl.ds(e0, 16)] for k_ in range(IN_F)]
            for j in range(OUT_F):
                acc = plsc.load_gather(b_v, [bb + j])
                for k_ in range(IN_F):
                    wv = plsc.load_gather(w_v, [wb + (j * IN_F + k_)])
                    acc = acc + xcols[k_] * wv
                x_v[j, pl.ds(e0, 16)] = jnp.maximum(acc, 0.0)
            return carry2

        def process(blk, ne):
            lo = blk * SC_CHE
            pltpu.sync_copy(xt_hbm.at[:, pl.ds(lo, ne)], x_v.at[:, pl.ds(0, ne)])
            lax.fori_loop(0, ne // 16, group_body, 0)
            pltpu.sync_copy(x_v.at[:, pl.ds(0, ne)], out_hbm.at[:, pl.ds(lo, ne)])

        def iter_body(i, carry):
            blk = wid + i * NW

            @pl.when(blk < SC_FULL)
            def _():
                pltpu.sync_copy(t_hbm.at[pl.ds(blk * SC_CHE, SC_CHE)], t_v)
                process(blk, SC_CHE)

            @pl.when(blk == SC_FULL)
            def _():
                pltpu.sync_copy(ttail_hbm, t_v.at[pl.ds(0, SC_TAIL)])
                process(blk, SC_TAIL)

            return carry

        lax.fori_loop(0, SC_NIT, iter_body, 0)

    return k(xt, types, t_tail, wflat, bflat)


@jax.jit
def kernel(edge_features, edge_types, W, b):
    xt = edge_features.T
    t_tail = lax.slice(edge_types, (E - SC_TAIL,), (E,))
    out_t = _sc_impl(xt, edge_types, t_tail, W.reshape(-1), b.reshape(-1))
    return out_t.T


@jax.jit
def _kernel_tc(edge_features, edge_types, W, b):
    xt = edge_features.T
    out_t = pl.pallas_call(
        _tc_body,
        grid=(NBLK,),
        in_specs=[
            pl.BlockSpec((IN_F, BLKE), lambda i: (0, i)),
            pl.BlockSpec((BLKE,), lambda i: (i,)),
            pl.BlockSpec((NUM_TYPES, OUT_F, IN_F), lambda i: (0, 0, 0)),
            pl.BlockSpec((NUM_TYPES, OUT_F), lambda i: (0, 0)),
        ],
        out_specs=pl.BlockSpec((OUT_F, BLKE), lambda i: (0, i)),
        out_shape=jax.ShapeDtypeStruct((OUT_F, E), jnp.float32),
    )(xt, edge_types, W, b)
    return out_t.T


# speedup vs baseline: 52.9800x; 52.9800x over previous
import functools

import jax
import jax.numpy as jnp
from jax import lax
from jax.experimental import pallas as pl
from jax.experimental.pallas import tpu as pltpu
from jax.experimental.pallas import tpu_sc as plsc

E = 1600000
IN_F = 16
OUT_F = 16
NUM_TYPES = 4

BLKE = 32768
NBLK = -(-E // BLKE)


def _tc_body(x_ref, t_ref, w_ref, b_ref, out_ref):
    x = x_ref[...]
    tw = t_ref[...][None, :]
    ys = []
    for tt in range(NUM_TYPES):
        y = lax.dot_general(
            w_ref[tt], x,
            dimension_numbers=(((1,), (0,)), ((), ())),
            preferred_element_type=jnp.float32,
        ) + b_ref[tt][:, None]
        ys.append(y)
    m0 = tw == 0
    m1 = tw == 1
    m2 = tw == 2
    ysel = jnp.where(m0, ys[0], jnp.where(m1, ys[1], jnp.where(m2, ys[2], ys[3])))
    out_ref[...] = jnp.maximum(ysel, 0.0)


SC_ROWS = 40
SC_CHE = SC_ROWS * 128
SC_FULL = E // SC_CHE
SC_TAIL = E - SC_FULL * SC_CHE
SC_NBLK = SC_FULL + 1
NW = 32
SC_NIT = -(-SC_NBLK // NW)


def _sc_impl(xt, types, t_tail, wflat, bflat):
    mesh = plsc.VectorSubcoreMesh(core_axis_name="c", subcore_axis_name="s")

    @functools.partial(
        pl.kernel,
        mesh=mesh,
        out_type=jax.ShapeDtypeStruct((OUT_F, E), jnp.float32),
        compiler_params=pltpu.CompilerParams(needs_layout_passes=False),
        scratch_types=[
            pltpu.VMEM((IN_F, SC_CHE), jnp.float32),
            pltpu.VMEM((SC_CHE,), jnp.int32),
            pltpu.VMEM((NUM_TYPES * OUT_F * IN_F,), jnp.float32),
            pltpu.VMEM((NUM_TYPES * OUT_F,), jnp.float32),
        ],
    )
    def k(xt_hbm, t_hbm, ttail_hbm, w_hbm, b_hbm, out_hbm, x_v, t_v, w_v, b_v):
        wid = lax.axis_index("s") * 2 + lax.axis_index("c")
        pltpu.sync_copy(w_hbm, w_v)
        pltpu.sync_copy(b_hbm, b_v)

        def group_body(g, carry2):
            e0 = g * 16
            tvec = t_v[pl.ds(e0, 16)]
            wb = tvec * (OUT_F * IN_F)
            bb = tvec * OUT_F
            xcols = [x_v[k_, pl.ds(e0, 16)] for k_ in range(IN_F)]
            for j in range(OUT_F):
                acc = plsc.load_gather(b_v, [bb + j])
                for k_ in range(IN_F):
                    wv = plsc.load_gather(w_v, [wb + (j * IN_F + k_)])
                    acc = acc + xcols[k_] * wv
                x_v[j, pl.ds(e0, 16)] = jnp.maximum(acc, 0.0)
            return carry2

        def process(blk, ne):
            lo = blk * SC_CHE
            pltpu.sync_copy(xt_hbm.at[:, pl.ds(lo, ne)], x_v.at[:, pl.ds(0, ne)])
            lax.fori_loop(0, ne // 16, group_body, 0)
            pltpu.sync_copy(x_v.at[:, pl.ds(0, ne)], out_hbm.at[:, pl.ds(lo, ne)])

        def iter_body(i, carry):
            blk = wid + i * NW

            @pl.when(blk < SC_FULL)
            def _():
                pltpu.sync_copy(t_hbm.at[pl.ds(blk * SC_CHE, SC_CHE)], t_v)
                process(blk, SC_CHE)

            @pl.when(blk == SC_FULL)
            def _():
                pltpu.sync_copy(ttail_hbm, t_v.at[pl.ds(0, SC_TAIL)])
                process(blk, SC_TAIL)

            return carry

        lax.fori_loop(0, SC_NIT, iter_body, 0)

    return k(xt, types, t_tail, wflat, bflat)


@jax.jit
def _kernel_sc(edge_features, edge_types, W, b):
    xt = edge_features.T
    t_tail = lax.slice(edge_types, (E - SC_TAIL,), (E,))
    out_t = _sc_impl(xt, edge_types, t_tail, W.reshape(-1), b.reshape(-1))
    return out_t.T


@jax.jit
def kernel(edge_features, edge_types, W, b):
    xt = edge_features.T
    out_t = pl.pallas_call(
        _tc_body,
        grid=(NBLK,),
        in_specs=[
            pl.BlockSpec((IN_F, BLKE), lambda i: (0, i)),
            pl.BlockSpec((BLKE,), lambda i: (i,)),
            pl.BlockSpec((NUM_TYPES, OUT_F, IN_F), lambda i: (0, 0, 0)),
            pl.BlockSpec((NUM_TYPES, OUT_F), lambda i: (0, 0)),
        ],
        out_specs=pl.BlockSpec((OUT_F, BLKE), lambda i: (0, i)),
        out_shape=jax.ShapeDtypeStruct((OUT_F, E), jnp.float32),
    )(xt, edge_types, W, b)
    return out_t.T


# speedup vs baseline: 59.1804x; 1.1170x over previous
import functools

import jax
import jax.numpy as jnp
from jax import lax
from jax.experimental import pallas as pl
from jax.experimental.pallas import tpu as pltpu
from jax.experimental.pallas import tpu_sc as plsc

E = 1600000
IN_F = 16
OUT_F = 16
NUM_TYPES = 4

BLKE = 65536
NBLK = -(-E // BLKE)


def _tc_body(x_ref, t_ref, w_ref, b_ref, out_ref):
    x = x_ref[...]
    tw = t_ref[...][None, :]
    ys = []
    for tt in range(NUM_TYPES):
        y = lax.dot_general(
            w_ref[tt], x,
            dimension_numbers=(((1,), (0,)), ((), ())),
            preferred_element_type=jnp.float32,
        ) + b_ref[tt][:, None]
        ys.append(y)
    m0 = tw == 0
    m1 = tw == 1
    m2 = tw == 2
    ysel = jnp.where(m0, ys[0], jnp.where(m1, ys[1], jnp.where(m2, ys[2], ys[3])))
    out_ref[...] = jnp.maximum(ysel, 0.0)


SC_ROWS = 40
SC_CHE = SC_ROWS * 128
SC_FULL = E // SC_CHE
SC_TAIL = E - SC_FULL * SC_CHE
SC_NBLK = SC_FULL + 1
NW = 32
SC_NIT = -(-SC_NBLK // NW)


def _sc_impl(xt, types, t_tail, wflat, bflat):
    mesh = plsc.VectorSubcoreMesh(core_axis_name="c", subcore_axis_name="s")

    @functools.partial(
        pl.kernel,
        mesh=mesh,
        out_type=jax.ShapeDtypeStruct((OUT_F, E), jnp.float32),
        compiler_params=pltpu.CompilerParams(needs_layout_passes=False),
        scratch_types=[
            pltpu.VMEM((IN_F, SC_CHE), jnp.float32),
            pltpu.VMEM((SC_CHE,), jnp.int32),
            pltpu.VMEM((NUM_TYPES * OUT_F * IN_F,), jnp.float32),
            pltpu.VMEM((NUM_TYPES * OUT_F,), jnp.float32),
        ],
    )
    def k(xt_hbm, t_hbm, ttail_hbm, w_hbm, b_hbm, out_hbm, x_v, t_v, w_v, b_v):
        wid = lax.axis_index("s") * 2 + lax.axis_index("c")
        pltpu.sync_copy(w_hbm, w_v)
        pltpu.sync_copy(b_hbm, b_v)

        def group_body(g, carry2):
            e0 = g * 16
            tvec = t_v[pl.ds(e0, 16)]
            wb = tvec * (OUT_F * IN_F)
            bb = tvec * OUT_F
            xcols = [x_v[k_, pl.ds(e0, 16)] for k_ in range(IN_F)]
            for j in range(OUT_F):
                acc = plsc.load_gather(b_v, [bb + j])
                for k_ in range(IN_F):
                    wv = plsc.load_gather(w_v, [wb + (j * IN_F + k_)])
                    acc = acc + xcols[k_] * wv
                x_v[j, pl.ds(e0, 16)] = jnp.maximum(acc, 0.0)
            return carry2

        def process(blk, ne):
            lo = blk * SC_CHE
            pltpu.sync_copy(xt_hbm.at[:, pl.ds(lo, ne)], x_v.at[:, pl.ds(0, ne)])
            lax.fori_loop(0, ne // 16, group_body, 0)
            pltpu.sync_copy(x_v.at[:, pl.ds(0, ne)], out_hbm.at[:, pl.ds(lo, ne)])

        def iter_body(i, carry):
            blk = wid + i * NW

            @pl.when(blk < SC_FULL)
            def _():
                pltpu.sync_copy(t_hbm.at[pl.ds(blk * SC_CHE, SC_CHE)], t_v)
                process(blk, SC_CHE)

            @pl.when(blk == SC_FULL)
            def _():
                pltpu.sync_copy(ttail_hbm, t_v.at[pl.ds(0, SC_TAIL)])
                process(blk, SC_TAIL)

            return carry

        lax.fori_loop(0, SC_NIT, iter_body, 0)

    return k(xt, types, t_tail, wflat, bflat)


@jax.jit
def _kernel_sc(edge_features, edge_types, W, b):
    xt = edge_features.T
    t_tail = lax.slice(edge_types, (E - SC_TAIL,), (E,))
    out_t = _sc_impl(xt, edge_types, t_tail, W.reshape(-1), b.reshape(-1))
    return out_t.T


@jax.jit
def kernel(edge_features, edge_types, W, b):
    xt = edge_features.T
    out_t = pl.pallas_call(
        _tc_body,
        grid=(NBLK,),
        in_specs=[
            pl.BlockSpec((IN_F, BLKE), lambda i: (0, i)),
            pl.BlockSpec((BLKE,), lambda i: (i,)),
            pl.BlockSpec((NUM_TYPES, OUT_F, IN_F), lambda i: (0, 0, 0)),
            pl.BlockSpec((NUM_TYPES, OUT_F), lambda i: (0, 0)),
        ],
        out_specs=pl.BlockSpec((OUT_F, BLKE), lambda i: (0, i)),
        out_shape=jax.ShapeDtypeStruct((OUT_F, E), jnp.float32),
    )(xt, edge_types, W, b)
    return out_t.T


# speedup vs baseline: 71.5528x; 1.2091x over previous
import functools

import jax
import jax.numpy as jnp
from jax import lax
from jax.experimental import pallas as pl
from jax.experimental.pallas import tpu as pltpu
from jax.experimental.pallas import tpu_sc as plsc

E = 1600000
IN_F = 16
OUT_F = 16
NUM_TYPES = 4

BLKE = 131072
NBLK = -(-E // BLKE)


def _tc_body(x_ref, t_ref, w_ref, b_ref, out_ref):
    x = x_ref[...]
    tw = t_ref[...][None, :]
    out = None
    for tt in range(NUM_TYPES):
        y = lax.dot_general(
            w_ref[tt], x,
            dimension_numbers=(((1,), (0,)), ((), ())),
            preferred_element_type=jnp.float32,
        ) + b_ref[tt][:, None]
        out = y if out is None else jnp.where(tw == tt, y, out)
    out_ref[...] = jnp.maximum(out, 0.0)


SC_ROWS = 40
SC_CHE = SC_ROWS * 128
SC_FULL = E // SC_CHE
SC_TAIL = E - SC_FULL * SC_CHE
SC_NBLK = SC_FULL + 1
NW = 32
SC_NIT = -(-SC_NBLK // NW)


def _sc_impl(xt, types, t_tail, wflat, bflat):
    mesh = plsc.VectorSubcoreMesh(core_axis_name="c", subcore_axis_name="s")

    @functools.partial(
        pl.kernel,
        mesh=mesh,
        out_type=jax.ShapeDtypeStruct((OUT_F, E), jnp.float32),
        compiler_params=pltpu.CompilerParams(needs_layout_passes=False),
        scratch_types=[
            pltpu.VMEM((IN_F, SC_CHE), jnp.float32),
            pltpu.VMEM((SC_CHE,), jnp.int32),
            pltpu.VMEM((NUM_TYPES * OUT_F * IN_F,), jnp.float32),
            pltpu.VMEM((NUM_TYPES * OUT_F,), jnp.float32),
        ],
    )
    def k(xt_hbm, t_hbm, ttail_hbm, w_hbm, b_hbm, out_hbm, x_v, t_v, w_v, b_v):
        wid = lax.axis_index("s") * 2 + lax.axis_index("c")
        pltpu.sync_copy(w_hbm, w_v)
        pltpu.sync_copy(b_hbm, b_v)

        def group_body(g, carry2):
            e0 = g * 16
            tvec = t_v[pl.ds(e0, 16)]
            wb = tvec * (OUT_F * IN_F)
            bb = tvec * OUT_F
            xcols = [x_v[k_, pl.ds(e0, 16)] for k_ in range(IN_F)]
            for j in range(OUT_F):
                acc = plsc.load_gather(b_v, [bb + j])
                for k_ in range(IN_F):
                    wv = plsc.load_gather(w_v, [wb + (j * IN_F + k_)])
                    acc = acc + xcols[k_] * wv
                x_v[j, pl.ds(e0, 16)] = jnp.maximum(acc, 0.0)
            return carry2

        def process(blk, ne):
            lo = blk * SC_CHE
            pltpu.sync_copy(xt_hbm.at[:, pl.ds(lo, ne)], x_v.at[:, pl.ds(0, ne)])
            lax.fori_loop(0, ne // 16, group_body, 0)
            pltpu.sync_copy(x_v.at[:, pl.ds(0, ne)], out_hbm.at[:, pl.ds(lo, ne)])

        def iter_body(i, carry):
            blk = wid + i * NW

            @pl.when(blk < SC_FULL)
            def _():
                pltpu.sync_copy(t_hbm.at[pl.ds(blk * SC_CHE, SC_CHE)], t_v)
                process(blk, SC_CHE)

            @pl.when(blk == SC_FULL)
            def _():
                pltpu.sync_copy(ttail_hbm, t_v.at[pl.ds(0, SC_TAIL)])
                process(blk, SC_TAIL)

            return carry

        lax.fori_loop(0, SC_NIT, iter_body, 0)

    return k(xt, types, t_tail, wflat, bflat)


@jax.jit
def _kernel_sc(edge_features, edge_types, W, b):
    xt = edge_features.T
    t_tail = lax.slice(edge_types, (E - SC_TAIL,), (E,))
    out_t = _sc_impl(xt, edge_types, t_tail, W.reshape(-1), b.reshape(-1))
    return out_t.T


@jax.jit
def kernel(edge_features, edge_types, W, b):
    xt = edge_features.T
    out_t = pl.pallas_call(
        _tc_body,
        grid=(NBLK,),
        in_specs=[
            pl.BlockSpec((IN_F, BLKE), lambda i: (0, i)),
            pl.BlockSpec((BLKE,), lambda i: (i,)),
            pl.BlockSpec((NUM_TYPES, OUT_F, IN_F), lambda i: (0, 0, 0)),
            pl.BlockSpec((NUM_TYPES, OUT_F), lambda i: (0, 0)),
        ],
        out_specs=pl.BlockSpec((OUT_F, BLKE), lambda i: (0, i)),
        out_shape=jax.ShapeDtypeStruct((OUT_F, E), jnp.float32),
    )(xt, edge_types, W, b)
    return out_t.T


# speedup vs baseline: 71.8219x; 1.0038x over previous
import functools

import jax
import jax.numpy as jnp
from jax import lax
from jax.experimental import pallas as pl
from jax.experimental.pallas import tpu as pltpu
from jax.experimental.pallas import tpu_sc as plsc

E = 1600000
IN_F = 16
OUT_F = 16
NUM_TYPES = 4

BLKE = 131072
NBLK = -(-E // BLKE)


def _tc_body(x_ref, t_ref, w_ref, b_ref, out_ref):
    x = x_ref[...].astype(jnp.bfloat16)
    tw = t_ref[...][None, :]
    out = None
    for tt in range(NUM_TYPES):
        y = lax.dot_general(
            w_ref[tt].astype(jnp.bfloat16), x,
            dimension_numbers=(((1,), (0,)), ((), ())),
            preferred_element_type=jnp.float32,
        ) + b_ref[tt][:, None]
        out = y if out is None else jnp.where(tw == tt, y, out)
    out_ref[...] = jnp.maximum(out, 0.0)


SC_ROWS = 40
SC_CHE = SC_ROWS * 128
SC_FULL = E // SC_CHE
SC_TAIL = E - SC_FULL * SC_CHE
SC_NBLK = SC_FULL + 1
NW = 32
SC_NIT = -(-SC_NBLK // NW)


def _sc_impl(xt, types, t_tail, wflat, bflat):
    mesh = plsc.VectorSubcoreMesh(core_axis_name="c", subcore_axis_name="s")

    @functools.partial(
        pl.kernel,
        mesh=mesh,
        out_type=jax.ShapeDtypeStruct((OUT_F, E), jnp.float32),
        compiler_params=pltpu.CompilerParams(needs_layout_passes=False),
        scratch_types=[
            pltpu.VMEM((IN_F, SC_CHE), jnp.float32),
            pltpu.VMEM((SC_CHE,), jnp.int32),
            pltpu.VMEM((NUM_TYPES * OUT_F * IN_F,), jnp.float32),
            pltpu.VMEM((NUM_TYPES * OUT_F,), jnp.float32),
        ],
    )
    def k(xt_hbm, t_hbm, ttail_hbm, w_hbm, b_hbm, out_hbm, x_v, t_v, w_v, b_v):
        wid = lax.axis_index("s") * 2 + lax.axis_index("c")
        pltpu.sync_copy(w_hbm, w_v)
        pltpu.sync_copy(b_hbm, b_v)

        def group_body(g, carry2):
            e0 = g * 16
            tvec = t_v[pl.ds(e0, 16)]
            wb = tvec * (OUT_F * IN_F)
            bb = tvec * OUT_F
            xcols = [x_v[k_, pl.ds(e0, 16)] for k_ in range(IN_F)]
            for j in range(OUT_F):
                acc = plsc.load_gather(b_v, [bb + j])
                for k_ in range(IN_F):
                    wv = plsc.load_gather(w_v, [wb + (j * IN_F + k_)])
                    acc = acc + xcols[k_] * wv
                x_v[j, pl.ds(e0, 16)] = jnp.maximum(acc, 0.0)
            return carry2

        def process(blk, ne):
            lo = blk * SC_CHE
            pltpu.sync_copy(xt_hbm.at[:, pl.ds(lo, ne)], x_v.at[:, pl.ds(0, ne)])
            lax.fori_loop(0, ne // 16, group_body, 0)
            pltpu.sync_copy(x_v.at[:, pl.ds(0, ne)], out_hbm.at[:, pl.ds(lo, ne)])

        def iter_body(i, carry):
            blk = wid + i * NW

            @pl.when(blk < SC_FULL)
            def _():
                pltpu.sync_copy(t_hbm.at[pl.ds(blk * SC_CHE, SC_CHE)], t_v)
                process(blk, SC_CHE)

            @pl.when(blk == SC_FULL)
            def _():
                pltpu.sync_copy(ttail_hbm, t_v.at[pl.ds(0, SC_TAIL)])
                process(blk, SC_TAIL)

            return carry

        lax.fori_loop(0, SC_NIT, iter_body, 0)

    return k(xt, types, t_tail, wflat, bflat)


@jax.jit
def _kernel_sc(edge_features, edge_types, W, b):
    xt = edge_features.T
    t_tail = lax.slice(edge_types, (E - SC_TAIL,), (E,))
    out_t = _sc_impl(xt, edge_types, t_tail, W.reshape(-1), b.reshape(-1))
    return out_t.T


@jax.jit
def kernel(edge_features, edge_types, W, b):
    xt = edge_features.T
    out_t = pl.pallas_call(
        _tc_body,
        grid=(NBLK,),
        in_specs=[
            pl.BlockSpec((IN_F, BLKE), lambda i: (0, i)),
            pl.BlockSpec((BLKE,), lambda i: (i,)),
            pl.BlockSpec((NUM_TYPES, OUT_F, IN_F), lambda i: (0, 0, 0)),
            pl.BlockSpec((NUM_TYPES, OUT_F), lambda i: (0, 0)),
        ],
        out_specs=pl.BlockSpec((OUT_F, BLKE), lambda i: (0, i)),
        out_shape=jax.ShapeDtypeStruct((OUT_F, E), jnp.float32),
    )(xt, edge_types, W, b)
    return out_t.T


# speedup vs baseline: 73.2428x; 1.0198x over previous
import functools

import jax
import jax.numpy as jnp
from jax import lax
from jax.experimental import pallas as pl
from jax.experimental.pallas import tpu as pltpu
from jax.experimental.pallas import tpu_sc as plsc

E = 1600000
IN_F = 16
OUT_F = 16
NUM_TYPES = 4

BLKE = 163840
NBLK = -(-E // BLKE)


def _tc_body(x_ref, t_ref, w_ref, b_ref, out_ref):
    x = x_ref[...]
    tw = t_ref[...][None, :]
    out = None
    for tt in range(NUM_TYPES):
        y = lax.dot_general(
            w_ref[tt], x,
            dimension_numbers=(((1,), (0,)), ((), ())),
            preferred_element_type=jnp.float32,
        ) + b_ref[tt][:, None]
        out = y if out is None else jnp.where(tw == tt, y, out)
    out_ref[...] = jnp.maximum(out, 0.0)


SC_ROWS = 40
SC_CHE = SC_ROWS * 128
SC_FULL = E // SC_CHE
SC_TAIL = E - SC_FULL * SC_CHE
SC_NBLK = SC_FULL + 1
NW = 32
SC_NIT = -(-SC_NBLK // NW)


def _sc_impl(xt, types, t_tail, wflat, bflat):
    mesh = plsc.VectorSubcoreMesh(core_axis_name="c", subcore_axis_name="s")

    @functools.partial(
        pl.kernel,
        mesh=mesh,
        out_type=jax.ShapeDtypeStruct((OUT_F, E), jnp.float32),
        compiler_params=pltpu.CompilerParams(needs_layout_passes=False),
        scratch_types=[
            pltpu.VMEM((IN_F, SC_CHE), jnp.float32),
            pltpu.VMEM((SC_CHE,), jnp.int32),
            pltpu.VMEM((NUM_TYPES * OUT_F * IN_F,), jnp.float32),
            pltpu.VMEM((NUM_TYPES * OUT_F,), jnp.float32),
        ],
    )
    def k(xt_hbm, t_hbm, ttail_hbm, w_hbm, b_hbm, out_hbm, x_v, t_v, w_v, b_v):
        wid = lax.axis_index("s") * 2 + lax.axis_index("c")
        pltpu.sync_copy(w_hbm, w_v)
        pltpu.sync_copy(b_hbm, b_v)

        def group_body(g, carry2):
            e0 = g * 16
            tvec = t_v[pl.ds(e0, 16)]
            wb = tvec * (OUT_F * IN_F)
            bb = tvec * OUT_F
            xcols = [x_v[k_, pl.ds(e0, 16)] for k_ in range(IN_F)]
            for j in range(OUT_F):
                acc = plsc.load_gather(b_v, [bb + j])
                for k_ in range(IN_F):
                    wv = plsc.load_gather(w_v, [wb + (j * IN_F + k_)])
                    acc = acc + xcols[k_] * wv
                x_v[j, pl.ds(e0, 16)] = jnp.maximum(acc, 0.0)
            return carry2

        def process(blk, ne):
            lo = blk * SC_CHE
            pltpu.sync_copy(xt_hbm.at[:, pl.ds(lo, ne)], x_v.at[:, pl.ds(0, ne)])
            lax.fori_loop(0, ne // 16, group_body, 0)
            pltpu.sync_copy(x_v.at[:, pl.ds(0, ne)], out_hbm.at[:, pl.ds(lo, ne)])

        def iter_body(i, carry):
            blk = wid + i * NW

            @pl.when(blk < SC_FULL)
            def _():
                pltpu.sync_copy(t_hbm.at[pl.ds(blk * SC_CHE, SC_CHE)], t_v)
                process(blk, SC_CHE)

            @pl.when(blk == SC_FULL)
            def _():
                pltpu.sync_copy(ttail_hbm, t_v.at[pl.ds(0, SC_TAIL)])
                process(blk, SC_TAIL)

            return carry

        lax.fori_loop(0, SC_NIT, iter_body, 0)

    return k(xt, types, t_tail, wflat, bflat)


@jax.jit
def _kernel_sc(edge_features, edge_types, W, b):
    xt = edge_features.T
    t_tail = lax.slice(edge_types, (E - SC_TAIL,), (E,))
    out_t = _sc_impl(xt, edge_types, t_tail, W.reshape(-1), b.reshape(-1))
    return out_t.T


@jax.jit
def kernel(edge_features, edge_types, W, b):
    xt = edge_features.T
    out_t = pl.pallas_call(
        _tc_body,
        grid=(NBLK,),
        in_specs=[
            pl.BlockSpec((IN_F, BLKE), lambda i: (0, i)),
            pl.BlockSpec((BLKE,), lambda i: (i,)),
            pl.BlockSpec((NUM_TYPES, OUT_F, IN_F), lambda i: (0, 0, 0)),
            pl.BlockSpec((NUM_TYPES, OUT_F), lambda i: (0, 0)),
        ],
        out_specs=pl.BlockSpec((OUT_F, BLKE), lambda i: (0, i)),
        out_shape=jax.ShapeDtypeStruct((OUT_F, E), jnp.float32),
    )(xt, edge_types, W, b)
    return out_t.T
